# baseline (speedup 1.0000x reference)
import functools

import jax
import jax.numpy as jnp
from jax import lax
from jax.experimental import pallas as pl
from jax.experimental.pallas import tpu as pltpu
from jax.experimental.pallas import tpu_sc as plsc

N = 10000
E = 160000
D = 256

NC = 2
NS = 16
NW = NC * NS
NP = 10240
NPT = NP // NW
ACCW = (NPT + 1) * D

C = 4000
G = 64

BM = 1024


def _mlp_body(h_ref, w1_ref, b1_ref, w2_ref, b2_ref, u_ref):
    t = jnp.dot(h_ref[...], w1_ref[...], preferred_element_type=jnp.float32)
    t = jnp.maximum(t + b1_ref[...], 0.0)
    u = jnp.dot(t, w2_ref[...], preferred_element_type=jnp.float32)
    u_ref[...] = u + b2_ref[...]


def _node_mlp(hp, w1, b1, w2, b2):
    return pl.pallas_call(
        _mlp_body,
        grid=(NP // BM,),
        in_specs=[
            pl.BlockSpec((BM, D), lambda i: (i, 0)),
            pl.BlockSpec((D, D), lambda i: (0, 0)),
            pl.BlockSpec((1, D), lambda i: (0, 0)),
            pl.BlockSpec((D, D), lambda i: (0, 0)),
            pl.BlockSpec((1, D), lambda i: (0, 0)),
        ],
        out_specs=pl.BlockSpec((BM, D), lambda i: (i, 0)),
        out_shape=jax.ShapeDtypeStruct((NP, D), jnp.float32),
    )(hp, w1, b1, w2, b2)


def _sc_body(src_hbm, dst_hbm, u_hbm, out_hbm,
             dstv, srcv, msrc, mdstl, gidx, rows, acc, sem):
    wid = lax.axis_index("s") * NC + lax.axis_index("c")
    lo = wid * NPT

    zeros16 = jnp.zeros((16,), jnp.float32)

    def zero_body(i, carry):
        for k in range(8):
            acc[pl.ds(i * 128 + k * 16, 16)] = zeros16
        return carry
    lax.fori_loop(0, ACCW // 128, zero_body, 0)

    iota = lax.iota(jnp.int32, 16)

    def chunk_body(ci, carry):
        base = pl.multiple_of(ci * C, C)
        pltpu.sync_copy(dst_hbm.at[pl.ds(base, C)], dstv)
        pltpu.sync_copy(src_hbm.at[pl.ds(base, C)], srcv)

        def scan_body(i, offv):
            d = dstv[pl.ds(i * 16, 16)]
            s = srcv[pl.ds(i * 16, 16)]
            m = (d >= lo) & (d < lo + NPT)
            c = jnp.cumsum(m.astype(jnp.int32))
            pos = offv + c - 1
            plsc.store_scatter(msrc, [pos], s, mask=m)
            plsc.store_scatter(mdstl, [pos], d - lo, mask=m)
            return offv + plsc.all_reduce_population_count(m)
        offv = lax.fori_loop(0, C // 16, scan_body,
                             jnp.zeros((16,), jnp.int32))
        nmatch = offv[0]

        for k in range(G // 16):
            idxv = nmatch + k * 16 + iota
            plsc.store_scatter(msrc, [idxv], lo + k * 16 + iota)
            plsc.store_scatter(mdstl, [idxv], jnp.full((16,), NPT, jnp.int32))

        nblocks = (nmatch + (G - 1)) // G

        def block_body(b, carry2):
            g = pl.multiple_of(b * G, G)
            for k in range(G // 16):
                gidx[pl.ds(k * 16, 16)] = msrc[pl.ds(g + k * 16, 16)]
            pltpu.async_copy(u_hbm.at[gidx], rows, sem).wait()

            def row_body(r, carry3):
                rbase = mdstl[pl.ds(g + r, 16)][0] * D
                rv = [rows[r, pl.ds(j * 16, 16)] for j in range(D // 16)]
                av = [acc[pl.ds(rbase + j * 16, 16)] for j in range(D // 16)]
                for j in range(D // 16):
                    acc[pl.ds(rbase + j * 16, 16)] = jnp.maximum(av[j], rv[j])
                return carry3
            lax.fori_loop(0, G, row_body, 0, unroll=4)
            return carry2
        lax.fori_loop(0, nblocks, block_body, 0)
        return carry
    lax.fori_loop(0, E // C, chunk_body, 0)

    pltpu.sync_copy(acc.at[pl.ds(0, NPT * D)],
                    out_hbm.at[pl.ds(lo * D, NPT * D)])


def _segment_max(src, dst, u):
    mesh = plsc.VectorSubcoreMesh(core_axis_name="c", subcore_axis_name="s")
    f = functools.partial(
        pl.kernel,
        out_type=jax.ShapeDtypeStruct((NP * D,), jnp.float32),
        mesh=mesh,
        scratch_types=[
            pltpu.VMEM((C,), jnp.int32),
            pltpu.VMEM((C,), jnp.int32),
            pltpu.VMEM((C + G + 16,), jnp.int32),
            pltpu.VMEM((C + G + 16,), jnp.int32),
            pltpu.VMEM((G,), jnp.int32),
            pltpu.VMEM((G, D), jnp.float32),
            pltpu.VMEM((ACCW,), jnp.float32),
            pltpu.SemaphoreType.DMA,
        ],
        compiler_params=pltpu.CompilerParams(needs_layout_passes=False),
    )(_sc_body)
    return f(src, dst, u)


def kernel(h, edge_index, W1, b1, W2, b2):
    src = edge_index[0]
    dst = edge_index[1]
    hp = jnp.pad(h, ((0, NP - N), (0, 0)))
    u = _node_mlp(hp, W1, b1.reshape(1, D), W2, b2.reshape(1, D))
    out_flat = _segment_max(src, dst, u)
    return out_flat.reshape(NP, D)[:N]


# speedup vs baseline: 1.0211x; 1.0211x over previous
import functools

import jax
import jax.numpy as jnp
from jax import lax
from jax.experimental import pallas as pl
from jax.experimental.pallas import tpu as pltpu
from jax.experimental.pallas import tpu_sc as plsc

N = 10000
E = 160000
D = 256

NC = 2
NS = 16
NW = NC * NS
NP = 10240
NPT = NP // NW
D2 = D // 2
ACCW = (NPT + 1) * D2

C = 8000
G = 128

BM = 1024


def _mlp_body(h_ref, w1_ref, b1_ref, w2_ref, b2_ref, u_ref):
    t = jnp.dot(h_ref[...], w1_ref[...], preferred_element_type=jnp.float32)
    t = jnp.maximum(t + b1_ref[...], 0.0)
    u = jnp.dot(t, w2_ref[...], preferred_element_type=jnp.float32)
    u_ref[...] = (u + b2_ref[...]).astype(jnp.bfloat16)


def _node_mlp(hp, w1, b1, w2, b2):
    return pl.pallas_call(
        _mlp_body,
        grid=(NP // BM,),
        in_specs=[
            pl.BlockSpec((BM, D), lambda i: (i, 0)),
            pl.BlockSpec((D, D), lambda i: (0, 0)),
            pl.BlockSpec((1, D), lambda i: (0, 0)),
            pl.BlockSpec((D, D), lambda i: (0, 0)),
            pl.BlockSpec((1, D), lambda i: (0, 0)),
        ],
        out_specs=pl.BlockSpec((BM, D), lambda i: (i, 0)),
        out_shape=jax.ShapeDtypeStruct((NP, D), jnp.bfloat16),
    )(hp, w1, b1, w2, b2)


def _sc_body(src_hbm, dst_hbm, u_hbm, out_hbm,
             dstv, srcv, msrc, mdstl, gidx, rows, acc, sem):
    wid = lax.axis_index("s") * NC + lax.axis_index("c")
    lo = wid * NPT

    zeros16 = jnp.zeros((16,), jnp.int32)

    def zero_body(i, carry):
        for k in range(8):
            acc[pl.ds(i * 128 + k * 16, 16)] = zeros16
        return carry
    lax.fori_loop(0, ACCW // 128, zero_body, 0)

    iota = lax.iota(jnp.int32, 16)

    def chunk_body(ci, carry):
        base = pl.multiple_of(ci * C, C)
        pltpu.sync_copy(dst_hbm.at[pl.ds(base, C)], dstv)
        pltpu.sync_copy(src_hbm.at[pl.ds(base, C)], srcv)

        def scan_body(i, offv):
            d = dstv[pl.ds(i * 16, 16)]
            s = srcv[pl.ds(i * 16, 16)]
            m = (d >= lo) & (d < lo + NPT)
            c = jnp.cumsum(m.astype(jnp.int32))
            pos = offv + c - 1
            plsc.store_scatter(msrc, [pos], s, mask=m)
            plsc.store_scatter(mdstl, [pos], d - lo, mask=m)
            return offv + plsc.all_reduce_population_count(m)
        offv = lax.fori_loop(0, C // 16, scan_body,
                             jnp.zeros((16,), jnp.int32), unroll=2)
        nmatch = offv[0]

        for k in range(G // 16):
            idxv = nmatch + k * 16 + iota
            plsc.store_scatter(msrc, [idxv], lo + k * 16 + iota)
            plsc.store_scatter(mdstl, [idxv], jnp.full((16,), NPT, jnp.int32))

        nblocks = (nmatch + (G - 1)) // G

        def block_body(b, carry2):
            g = pl.multiple_of(b * G, G)
            for k in range(G // 16):
                gidx[pl.ds(k * 16, 16)] = msrc[pl.ds(g + k * 16, 16)]
            pltpu.async_copy(u_hbm.at[gidx], rows, sem).wait()

            def row_body(r, carry3):
                rbase = mdstl[pl.ds(g + r, 16)][0] * D2
                nj = D2 // 16
                rv = [plsc.bitcast(rows[r, pl.ds(j * 16, 16)], jnp.bfloat16)
                      for j in range(nj)]
                av = [plsc.bitcast(acc[pl.ds(rbase + j * 16, 16)],
                                   jnp.bfloat16) for j in range(nj)]
                for j in range(nj):
                    acc[pl.ds(rbase + j * 16, 16)] = plsc.bitcast(
                        jnp.maximum(av[j], rv[j]), jnp.int32)
                return carry3
            lax.fori_loop(0, G, row_body, 0, unroll=4)
            return carry2
        lax.fori_loop(0, nblocks, block_body, 0)
        return carry
    lax.fori_loop(0, E // C, chunk_body, 0)

    pltpu.sync_copy(acc.at[pl.ds(0, NPT * D2)],
                    out_hbm.at[pl.ds(lo * D2, NPT * D2)])


def _segment_max(src, dst, u):
    mesh = plsc.VectorSubcoreMesh(core_axis_name="c", subcore_axis_name="s")
    f = functools.partial(
        pl.kernel,
        out_type=jax.ShapeDtypeStruct((NP * D2,), jnp.int32),
        mesh=mesh,
        scratch_types=[
            pltpu.VMEM((C,), jnp.int32),
            pltpu.VMEM((C,), jnp.int32),
            pltpu.VMEM((C + G + 16,), jnp.int32),
            pltpu.VMEM((C + G + 16,), jnp.int32),
            pltpu.VMEM((G,), jnp.int32),
            pltpu.VMEM((G, D2), jnp.int32),
            pltpu.VMEM((ACCW,), jnp.int32),
            pltpu.SemaphoreType.DMA,
        ],
        compiler_params=pltpu.CompilerParams(needs_layout_passes=False),
    )(_sc_body)
    return f(src, dst, u)


def kernel(h, edge_index, W1, b1, W2, b2):
    src = edge_index[0]
    dst = edge_index[1]
    hp = jnp.pad(h, ((0, NP - N), (0, 0)))
    u = _node_mlp(hp, W1, b1.reshape(1, D), W2, b2.reshape(1, D))
    u32 = jax.lax.bitcast_convert_type(u.reshape(NP, D2, 2), jnp.int32)
    out32 = _segment_max(src, dst, u32)
    ob = jax.lax.bitcast_convert_type(out32.reshape(NP, D2), jnp.bfloat16)
    return ob.reshape(NP, D)[:N].astype(jnp.float32)


# speedup vs baseline: 1.1416x; 1.1180x over previous
import functools

import jax
import jax.numpy as jnp
from jax import lax
from jax.experimental import pallas as pl
from jax.experimental.pallas import tpu as pltpu
from jax.experimental.pallas import tpu_sc as plsc

N = 10000
E = 160000
D = 256

NC = 2
NS = 16
NW = NC * NS
NP = 10240
NPT = NP // NW
D2 = D // 2
ACCW = (NPT + 1) * D2

C = 8000
G = 128

BM = 1000


def _mlp_body(h_ref, w1_ref, b1_ref, w2e_ref, w2o_ref, b2e_ref, b2o_ref,
              u_ref):
    t = jnp.dot(h_ref[...], w1_ref[...], preferred_element_type=jnp.float32)
    t = jnp.maximum(t + b1_ref[...], 0.0)
    ue = jnp.dot(t, w2e_ref[...], preferred_element_type=jnp.float32)
    uo = jnp.dot(t, w2o_ref[...], preferred_element_type=jnp.float32)
    pe = lax.bitcast_convert_type((ue + b2e_ref[...]).astype(jnp.bfloat16),
                                  jnp.uint16).astype(jnp.uint32)
    po = lax.bitcast_convert_type((uo + b2o_ref[...]).astype(jnp.bfloat16),
                                  jnp.uint16).astype(jnp.uint32)
    u_ref[...] = lax.bitcast_convert_type(pe | (po << 16), jnp.int32)


def _node_mlp(hp, w1, b1, w2e, w2o, b2e, b2o):
    return pl.pallas_call(
        _mlp_body,
        grid=(NP // BM,),
        in_specs=[
            pl.BlockSpec((BM, D), lambda i: (i, 0)),
            pl.BlockSpec((D, D), lambda i: (0, 0)),
            pl.BlockSpec((1, D), lambda i: (0, 0)),
            pl.BlockSpec((D, D2), lambda i: (0, 0)),
            pl.BlockSpec((D, D2), lambda i: (0, 0)),
            pl.BlockSpec((1, D2), lambda i: (0, 0)),
            pl.BlockSpec((1, D2), lambda i: (0, 0)),
        ],
        out_specs=pl.BlockSpec((BM, D2), lambda i: (i, 0)),
        out_shape=jax.ShapeDtypeStruct((N, D2), jnp.int32),
    )(hp, w1, b1, w2e, w2o, b2e, b2o)


def _sc_body(src_hbm, dst_hbm, u_hbm, out_hbm,
             dstv, srcv, msrc, mdstl, gidx, rows, acc, sem):
    wid = lax.axis_index("s") * NC + lax.axis_index("c")
    lo = wid * NPT

    zeros16 = jnp.zeros((16,), jnp.int32)

    def zero_body(i, carry):
        for k in range(8):
            acc[pl.ds(i * 128 + k * 16, 16)] = zeros16
        return carry
    lax.fori_loop(0, ACCW // 128, zero_body, 0)

    iota = lax.iota(jnp.int32, 16)

    def chunk_body(ci, carry):
        base = pl.multiple_of(ci * C, C)
        pltpu.sync_copy(dst_hbm.at[pl.ds(base, C)], dstv)
        pltpu.sync_copy(src_hbm.at[pl.ds(base, C)], srcv)

        def scan_body(i, offv):
            d = dstv[pl.ds(i * 16, 16)]
            s = srcv[pl.ds(i * 16, 16)]
            m = (d >= lo) & (d < lo + NPT)
            c = jnp.cumsum(m.astype(jnp.int32))
            pos = offv + c - 1
            plsc.store_scatter(msrc, [pos], s, mask=m)
            plsc.store_scatter(mdstl, [pos], d - lo, mask=m)
            return offv + plsc.all_reduce_population_count(m)
        offv = lax.fori_loop(0, C // 16, scan_body,
                             jnp.zeros((16,), jnp.int32), unroll=2)
        nmatch = offv[0]

        for k in range(G // 16):
            idxv = nmatch + k * 16 + iota
            plsc.store_scatter(msrc, [idxv], lo + k * 16 + iota)
            plsc.store_scatter(mdstl, [idxv], jnp.full((16,), NPT, jnp.int32))

        nblocks = (nmatch + (G - 1)) // G

        def block_body(b, carry2):
            g = pl.multiple_of(b * G, G)
            for k in range(G // 16):
                gidx[pl.ds(k * 16, 16)] = msrc[pl.ds(g + k * 16, 16)]
            pltpu.async_copy(u_hbm.at[gidx], rows, sem).wait()

            def row_body(r, carry3):
                rbase = mdstl[pl.ds(g + r, 16)][0] * D2
                nj = D2 // 16
                rv = [plsc.bitcast(rows[r, pl.ds(j * 16, 16)], jnp.bfloat16)
                      for j in range(nj)]
                av = [plsc.bitcast(acc[pl.ds(rbase + j * 16, 16)],
                                   jnp.bfloat16) for j in range(nj)]
                for j in range(nj):
                    acc[pl.ds(rbase + j * 16, 16)] = plsc.bitcast(
                        jnp.maximum(av[j], rv[j]), jnp.int32)
                return carry3
            lax.fori_loop(0, G, row_body, 0, unroll=4)
            return carry2
        lax.fori_loop(0, nblocks, block_body, 0)
        return carry
    lax.fori_loop(0, E // C, chunk_body, 0)

    pltpu.sync_copy(acc.at[pl.ds(0, NPT * D2)],
                    out_hbm.at[pl.ds(lo * D2, NPT * D2)])


def _segment_max(src, dst, u):
    mesh = plsc.VectorSubcoreMesh(core_axis_name="c", subcore_axis_name="s")
    f = functools.partial(
        pl.kernel,
        out_type=jax.ShapeDtypeStruct((NP * D2,), jnp.int32),
        mesh=mesh,
        scratch_types=[
            pltpu.VMEM((C,), jnp.int32),
            pltpu.VMEM((C,), jnp.int32),
            pltpu.VMEM((C + G + 16,), jnp.int32),
            pltpu.VMEM((C + G + 16,), jnp.int32),
            pltpu.VMEM((G,), jnp.int32),
            pltpu.VMEM((G, D2), jnp.int32),
            pltpu.VMEM((ACCW,), jnp.int32),
            pltpu.SemaphoreType.DMA,
        ],
        compiler_params=pltpu.CompilerParams(needs_layout_passes=False),
    )(_sc_body)
    return f(src, dst, u)


def kernel(h, edge_index, W1, b1, W2, b2):
    src = edge_index[0]
    dst = edge_index[1]
    u32 = _node_mlp(h, W1, b1.reshape(1, D), W2[:, 0::2], W2[:, 1::2],
                    b2[0::2].reshape(1, D2), b2[1::2].reshape(1, D2))
    out32 = _segment_max(src, dst, u32)
    ob = jax.lax.bitcast_convert_type(out32.reshape(NP, D2), jnp.bfloat16)
    return ob.reshape(NP, D)[:N].astype(jnp.float32)


# speedup vs baseline: 1.2202x; 1.0689x over previous
import functools

import jax
import jax.numpy as jnp
from jax import lax
from jax.experimental import pallas as pl
from jax.experimental.pallas import tpu as pltpu
from jax.experimental.pallas import tpu_sc as plsc

N = 10000
E = 160000
D = 256

NC = 2
NS = 16
NW = NC * NS
NP = 10240
NR = 8
NPT = NP // 2 // NR
D2 = D // 2
ACCW = (NPT + 1) * D2
EH = E // 2

C = 4000
G = 128
STRW = 8192

BM = 1000


def _mlp_body(h_ref, w1_ref, b1_ref, w2e_ref, w2o_ref, b2e_ref, b2o_ref,
              u_ref):
    t = jnp.dot(h_ref[...], w1_ref[...], preferred_element_type=jnp.float32)
    t = jnp.maximum(t + b1_ref[...], 0.0)
    ue = jnp.dot(t, w2e_ref[...], preferred_element_type=jnp.float32)
    uo = jnp.dot(t, w2o_ref[...], preferred_element_type=jnp.float32)
    pe = lax.bitcast_convert_type((ue + b2e_ref[...]).astype(jnp.bfloat16),
                                  jnp.uint16).astype(jnp.uint32)
    po = lax.bitcast_convert_type((uo + b2o_ref[...]).astype(jnp.bfloat16),
                                  jnp.uint16).astype(jnp.uint32)
    u_ref[...] = lax.bitcast_convert_type(pe | (po << 16), jnp.int32)


def _node_mlp(hp, w1, b1, w2e, w2o, b2e, b2o):
    return pl.pallas_call(
        _mlp_body,
        grid=(NP // BM,),
        in_specs=[
            pl.BlockSpec((BM, D), lambda i: (i, 0)),
            pl.BlockSpec((D, D), lambda i: (0, 0)),
            pl.BlockSpec((1, D), lambda i: (0, 0)),
            pl.BlockSpec((D, D2), lambda i: (0, 0)),
            pl.BlockSpec((D, D2), lambda i: (0, 0)),
            pl.BlockSpec((1, D2), lambda i: (0, 0)),
            pl.BlockSpec((1, D2), lambda i: (0, 0)),
        ],
        out_specs=pl.BlockSpec((BM, D2), lambda i: (i, 0)),
        out_shape=jax.ShapeDtypeStruct((N, D2), jnp.int32),
    )(hp, w1, b1, w2e, w2o, b2e, b2o)


def _sc_body(src_hbm, dst_hbm, u_hbm, out_hbm,
             dstv, srcv, msrc, mdstl, gidx, rows, strip, acc, shared, sem):
    core = lax.axis_index("c")
    sub = lax.axis_index("s")
    rng = sub % NR
    half = sub // NR
    lo = core * (NR * NPT) + rng * NPT

    zeros16 = jnp.zeros((16,), jnp.int32)

    def zero_body(i, carry):
        for k in range(8):
            acc[pl.ds(i * 128 + k * 16, 16)] = zeros16
        return carry
    lax.fori_loop(0, ACCW // 128, zero_body, 0)

    iota = lax.iota(jnp.int32, 16)

    def chunk_body(ci, carry):
        base = pl.multiple_of(half * EH + ci * C, C)
        pltpu.sync_copy(dst_hbm.at[pl.ds(base, C)], dstv)
        pltpu.sync_copy(src_hbm.at[pl.ds(base, C)], srcv)

        def scan_body(i, offv):
            d = dstv[pl.ds(i * 16, 16)]
            s = srcv[pl.ds(i * 16, 16)]
            m = (d >= lo) & (d < lo + NPT)
            c = jnp.cumsum(m.astype(jnp.int32))
            pos = offv + c - 1
            plsc.store_scatter(msrc, [pos], s, mask=m)
            plsc.store_scatter(mdstl, [pos], d - lo, mask=m)
            return offv + plsc.all_reduce_population_count(m)
        offv = lax.fori_loop(0, C // 16, scan_body,
                             jnp.zeros((16,), jnp.int32), unroll=2)
        nmatch = offv[0]

        for k in range(G // 16):
            idxv = nmatch + k * 16 + iota
            plsc.store_scatter(msrc, [idxv], lo + k * 16 + iota)
            plsc.store_scatter(mdstl, [idxv], jnp.full((16,), NPT, jnp.int32))

        nblocks = (nmatch + (G - 1)) // G

        def block_body(b, carry2):
            g = pl.multiple_of(b * G, G)
            for k in range(G // 16):
                gidx[pl.ds(k * 16, 16)] = msrc[pl.ds(g + k * 16, 16)]
            pltpu.async_copy(u_hbm.at[gidx], rows, sem).wait()

            def row_body(r, carry3):
                rbase = mdstl[pl.ds(g + r, 16)][0] * D2
                nj = D2 // 16
                rv = [plsc.bitcast(rows[r, pl.ds(j * 16, 16)], jnp.bfloat16)
                      for j in range(nj)]
                av = [plsc.bitcast(acc[pl.ds(rbase + j * 16, 16)],
                                   jnp.bfloat16) for j in range(nj)]
                for j in range(nj):
                    acc[pl.ds(rbase + j * 16, 16)] = plsc.bitcast(
                        jnp.maximum(av[j], rv[j]), jnp.int32)
                return carry3
            lax.fori_loop(0, G, row_body, 0, unroll=4)
            return carry2
        lax.fori_loop(0, nblocks, block_body, 0)
        return carry
    lax.fori_loop(0, EH // C, chunk_body, 0)

    def comb_body(k, carry):
        kb = pl.multiple_of(k * STRW, STRW)

        @pl.when(half == 1)
        def _publish():
            pltpu.sync_copy(acc.at[pl.ds(kb, STRW)], shared.at[rng])
        plsc.subcore_barrier()

        @pl.when(half == 0)
        def _merge():
            pltpu.sync_copy(shared.at[rng], strip)

            def merge_body(i, carry2):
                for k2 in range(8):
                    o = i * 128 + k2 * 16
                    a = plsc.bitcast(acc[pl.ds(kb + o, 16)], jnp.bfloat16)
                    b = plsc.bitcast(strip[pl.ds(o, 16)], jnp.bfloat16)
                    acc[pl.ds(kb + o, 16)] = plsc.bitcast(
                        jnp.maximum(a, b), jnp.int32)
                return carry2
            lax.fori_loop(0, STRW // 128, merge_body, 0)
        plsc.subcore_barrier()
        return carry
    lax.fori_loop(0, NPT * D2 // STRW, comb_body, 0)

    @pl.when(half == 0)
    def _writeout():
        pltpu.sync_copy(acc.at[pl.ds(0, NPT * D2)],
                        out_hbm.at[pl.ds(lo * D2, NPT * D2)])


def _segment_max(src, dst, u):
    mesh = plsc.VectorSubcoreMesh(core_axis_name="c", subcore_axis_name="s")
    f = functools.partial(
        pl.kernel,
        out_type=jax.ShapeDtypeStruct((NP * D2,), jnp.int32),
        mesh=mesh,
        scratch_types=[
            pltpu.VMEM((C,), jnp.int32),
            pltpu.VMEM((C,), jnp.int32),
            pltpu.VMEM((C + G + 16,), jnp.int32),
            pltpu.VMEM((C + G + 16,), jnp.int32),
            pltpu.VMEM((G,), jnp.int32),
            pltpu.VMEM((G, D2), jnp.int32),
            pltpu.VMEM((STRW,), jnp.int32),
            pltpu.VMEM((ACCW,), jnp.int32),
            pltpu.VMEM_SHARED((NR, STRW), jnp.int32),
            pltpu.SemaphoreType.DMA,
        ],
        compiler_params=pltpu.CompilerParams(needs_layout_passes=False),
    )(_sc_body)
    return f(src, dst, u)


def kernel(h, edge_index, W1, b1, W2, b2):
    src = edge_index[0]
    dst = edge_index[1]
    u32 = _node_mlp(h, W1, b1.reshape(1, D), W2[:, 0::2], W2[:, 1::2],
                    b2[0::2].reshape(1, D2), b2[1::2].reshape(1, D2))
    out32 = _segment_max(src, dst, u32)
    ob = jax.lax.bitcast_convert_type(out32.reshape(NP, D2), jnp.bfloat16)
    return ob.reshape(NP, D)[:N].astype(jnp.float32)


# speedup vs baseline: 1.3579x; 1.1129x over previous
import functools

import jax
import jax.numpy as jnp
from jax import lax
from jax.experimental import pallas as pl
from jax.experimental.pallas import tpu as pltpu
from jax.experimental.pallas import tpu_sc as plsc

N = 10000
E = 160000
D = 256

NC = 2
NS = 16
NW = NC * NS
NP = 10240
NR = 8
NPT = NP // 2 // NR
D2 = D // 2
ACCW = (NPT + 1) * D2
EH = E // 2

C = 3200
G = 96
STRW = 4096

BM = 1000


def _mlp_body(h_ref, w1_ref, b1_ref, w2e_ref, w2o_ref, b2e_ref, b2o_ref,
              u_ref):
    t = jnp.dot(h_ref[...], w1_ref[...], preferred_element_type=jnp.float32)
    t = jnp.maximum(t + b1_ref[...], 0.0)
    ue = jnp.dot(t, w2e_ref[...], preferred_element_type=jnp.float32)
    uo = jnp.dot(t, w2o_ref[...], preferred_element_type=jnp.float32)
    pe = lax.bitcast_convert_type((ue + b2e_ref[...]).astype(jnp.bfloat16),
                                  jnp.uint16).astype(jnp.uint32)
    po = lax.bitcast_convert_type((uo + b2o_ref[...]).astype(jnp.bfloat16),
                                  jnp.uint16).astype(jnp.uint32)
    u_ref[...] = lax.bitcast_convert_type(pe | (po << 16), jnp.int32)


def _node_mlp(hp, w1, b1, w2e, w2o, b2e, b2o):
    return pl.pallas_call(
        _mlp_body,
        grid=(NP // BM,),
        in_specs=[
            pl.BlockSpec((BM, D), lambda i: (i, 0)),
            pl.BlockSpec((D, D), lambda i: (0, 0)),
            pl.BlockSpec((1, D), lambda i: (0, 0)),
            pl.BlockSpec((D, D2), lambda i: (0, 0)),
            pl.BlockSpec((D, D2), lambda i: (0, 0)),
            pl.BlockSpec((1, D2), lambda i: (0, 0)),
            pl.BlockSpec((1, D2), lambda i: (0, 0)),
        ],
        out_specs=pl.BlockSpec((BM, D2), lambda i: (i, 0)),
        out_shape=jax.ShapeDtypeStruct((N, D2), jnp.int32),
    )(hp, w1, b1, w2e, w2o, b2e, b2o)


def _sc_body(src_hbm, dst_hbm, u_hbm, out_hbm,
             dstv, srcv, msrc, mdstl, gidx0, gidx1, rows0, rows1,
             strip, acc, shared, sem0, sem1, sema, semb):
    core = lax.axis_index("c")
    sub = lax.axis_index("s")
    rng = sub % NR
    half = sub // NR
    lo = core * (NR * NPT) + rng * NPT

    zeros16 = jnp.zeros((16,), jnp.int32)

    def zero_body(i, carry):
        for k in range(8):
            acc[pl.ds(i * 128 + k * 16, 16)] = zeros16
        return carry
    lax.fori_loop(0, ACCW // 128, zero_body, 0)

    iota = lax.iota(jnp.int32, 16)

    def chunk_body(ci, carry):
        base = pl.multiple_of(half * EH + ci * C, C)
        cpd = pltpu.async_copy(dst_hbm.at[pl.ds(base, C)], dstv, sema)
        cps = pltpu.async_copy(src_hbm.at[pl.ds(base, C)], srcv, semb)
        cpd.wait()
        cps.wait()

        def scan_body(i, offv):
            d = dstv[pl.ds(i * 16, 16)]
            s = srcv[pl.ds(i * 16, 16)]
            m = (d >= lo) & (d < lo + NPT)
            c = jnp.cumsum(m.astype(jnp.int32))
            pos = offv + c - 1
            plsc.store_scatter(msrc, [pos], s, mask=m)
            plsc.store_scatter(mdstl, [pos], d - lo, mask=m)
            return offv + plsc.all_reduce_population_count(m)
        offv = lax.fori_loop(0, C // 16, scan_body,
                             jnp.zeros((16,), jnp.int32), unroll=2)
        nmatch = offv[0]

        for k in range(G // 16):
            idxv = nmatch + k * 16 + iota
            plsc.store_scatter(msrc, [idxv], lo + k * 16 + iota)
            plsc.store_scatter(mdstl, [idxv], jnp.full((16,), NPT, jnp.int32))

        nblocks = (nmatch + (G - 1)) // G

        def fill_and_start(b, gidx, rows, sem):
            g = pl.multiple_of(b * G, G)
            for k in range(G // 16):
                gidx[pl.ds(k * 16, 16)] = msrc[pl.ds(g + k * 16, 16)]
            pltpu.async_copy(u_hbm.at[gidx], rows, sem)

        def accumulate(b, rows):
            g = pl.multiple_of(b * G, G)

            def row_body(r, carry3):
                rbase = mdstl[pl.ds(g + r, 16)][0] * D2
                nj = D2 // 16
                rv = [plsc.bitcast(rows[r, pl.ds(j * 16, 16)], jnp.bfloat16)
                      for j in range(nj)]
                av = [plsc.bitcast(acc[pl.ds(rbase + j * 16, 16)],
                                   jnp.bfloat16) for j in range(nj)]
                for j in range(nj):
                    acc[pl.ds(rbase + j * 16, 16)] = plsc.bitcast(
                        jnp.maximum(av[j], rv[j]), jnp.int32)
                return carry3
            lax.fori_loop(0, G, row_body, 0, unroll=4)

        @pl.when(nblocks > 0)
        def _prime():
            fill_and_start(0, gidx0, rows0, sem0)

        def block_body(b, carry2):
            even = (b % 2) == 0

            @pl.when(even)
            def _even():
                pltpu.make_async_copy(u_hbm.at[gidx0], rows0, sem0).wait()

                @pl.when(b + 1 < nblocks)
                def _next():
                    fill_and_start(b + 1, gidx1, rows1, sem1)
                accumulate(b, rows0)

            @pl.when(jnp.logical_not(even))
            def _odd():
                pltpu.make_async_copy(u_hbm.at[gidx1], rows1, sem1).wait()

                @pl.when(b + 1 < nblocks)
                def _next():
                    fill_and_start(b + 1, gidx0, rows0, sem0)
                accumulate(b, rows1)
            return carry2
        lax.fori_loop(0, nblocks, block_body, 0)
        return carry
    lax.fori_loop(0, EH // C, chunk_body, 0)

    def comb_body(k, carry):
        kb = pl.multiple_of(k * STRW, STRW)

        @pl.when(half == 1)
        def _publish():
            pltpu.sync_copy(acc.at[pl.ds(kb, STRW)], shared.at[rng])
        plsc.subcore_barrier()

        @pl.when(half == 0)
        def _merge():
            pltpu.sync_copy(shared.at[rng], strip)

            def merge_body(i, carry2):
                for k2 in range(8):
                    o = i * 128 + k2 * 16
                    a = plsc.bitcast(acc[pl.ds(kb + o, 16)], jnp.bfloat16)
                    b = plsc.bitcast(strip[pl.ds(o, 16)], jnp.bfloat16)
                    acc[pl.ds(kb + o, 16)] = plsc.bitcast(
                        jnp.maximum(a, b), jnp.int32)
                return carry2
            lax.fori_loop(0, STRW // 128, merge_body, 0)
        plsc.subcore_barrier()
        return carry
    lax.fori_loop(0, NPT * D2 // STRW, comb_body, 0)

    @pl.when(half == 0)
    def _writeout():
        pltpu.sync_copy(acc.at[pl.ds(0, NPT * D2)],
                        out_hbm.at[pl.ds(lo * D2, NPT * D2)])


def _segment_max(src, dst, u):
    mesh = plsc.VectorSubcoreMesh(core_axis_name="c", subcore_axis_name="s")
    f = functools.partial(
        pl.kernel,
        out_type=jax.ShapeDtypeStruct((NP * D2,), jnp.int32),
        mesh=mesh,
        scratch_types=[
            pltpu.VMEM((C,), jnp.int32),
            pltpu.VMEM((C,), jnp.int32),
            pltpu.VMEM((C + G + 16,), jnp.int32),
            pltpu.VMEM((C + G + 16,), jnp.int32),
            pltpu.VMEM((G,), jnp.int32),
            pltpu.VMEM((G,), jnp.int32),
            pltpu.VMEM((G, D2), jnp.int32),
            pltpu.VMEM((G, D2), jnp.int32),
            pltpu.VMEM((STRW,), jnp.int32),
            pltpu.VMEM((ACCW,), jnp.int32),
            pltpu.VMEM_SHARED((NR, STRW), jnp.int32),
            pltpu.SemaphoreType.DMA,
            pltpu.SemaphoreType.DMA,
            pltpu.SemaphoreType.DMA,
            pltpu.SemaphoreType.DMA,
        ],
        compiler_params=pltpu.CompilerParams(needs_layout_passes=False),
    )(_sc_body)
    return f(src, dst, u)


def kernel(h, edge_index, W1, b1, W2, b2):
    src = edge_index[0]
    dst = edge_index[1]
    u32 = _node_mlp(h, W1, b1.reshape(1, D), W2[:, 0::2], W2[:, 1::2],
                    b2[0::2].reshape(1, D2), b2[1::2].reshape(1, D2))
    out32 = _segment_max(src, dst, u32)
    ob = jax.lax.bitcast_convert_type(out32.reshape(NP, D2), jnp.bfloat16)
    return ob.reshape(NP, D)[:N].astype(jnp.float32)
